# baseline (speedup 1.0000x reference)
import functools

import jax
import jax.numpy as jnp
import numpy as np
from jax import lax
from jax.experimental import pallas as pl
from jax.experimental.pallas import tpu as pltpu
from jax.experimental.pallas import tpu_sc as plsc

N = 10000
D = 128
K = 3
D_ATTR = 16
EPS = 1e-15

NC = 2
NS = 16
NW = NC * NS
E_PER_W = 10240
E_PAD = NW * E_PER_W
R = 10240
CROWS = R // D
ACC_R = R + CROWS + 48
ROWS_PER_TILE = ACC_R // NS
TRASH = N + 50
WXI = 32
BCH = 8
GW = 256

PV = np.concatenate(
    [32 * j + np.concatenate([np.arange(0, 32, 2), np.arange(1, 32, 2)])
     for j in range(4)]).astype(np.int32)
IPV = np.argsort(PV).astype(np.int32)


def _pack_cols(xg):
    b = xg.shape[0]
    return jnp.concatenate(
        [xg.astype(jnp.bfloat16),
         jnp.zeros((b, GW * 2 - K * D), jnp.bfloat16)], axis=1)


def _as_i32(xgp):
    return lax.bitcast_convert_type(xgp.reshape(R, GW, 2), jnp.int32)


def _wx_i32(wx):
    return lax.bitcast_convert_type(
        wx.reshape(E_PAD, WXI, 2), jnp.int32).reshape(-1)


def _w_body(attr_ref, mu_ref, s_ref, wx_ref):
    a = attr_ref[...]
    b = a.shape[0]
    ws = []
    for k in range(K):
        mu = mu_ref[k, :]
        s2 = EPS + s_ref[k, :] ** 2
        g = -0.5 * (a - mu[None, :]) ** 2 / s2[None, :]
        ws.append(jnp.exp(jnp.sum(g, axis=1)).astype(jnp.bfloat16))
    h1 = jnp.broadcast_to(
        jnp.stack([ws[0], ws[1]], axis=-1)[:, None, :], (b, 16, 2))
    h2 = jnp.broadcast_to(
        jnp.stack([ws[2], ws[2]], axis=-1)[:, None, :], (b, 16, 2))
    wx_ref[...] = jnp.concatenate(
        [h1.reshape(b, 32), h2.reshape(b, 32)], axis=1)


def _edge_weights(edge_attr_p, mu, sigma):
    blk = 4096
    grid = E_PAD // blk
    small = pl.BlockSpec((K, D_ATTR), lambda i: (0, 0))
    return pl.pallas_call(
        _w_body,
        grid=(grid,),
        in_specs=[pl.BlockSpec((blk, D_ATTR), lambda i: (i, 0)),
                  small, small],
        out_specs=pl.BlockSpec((blk, WXI * 2), lambda i: (i, 0)),
        out_shape=jax.ShapeDtypeStruct((E_PAD, WXI * 2), jnp.bfloat16),
    )(edge_attr_p, mu, sigma)


def _pre_body(x_ref, g_ref, root_ref, xg_ref, r_ref):
    xb = x_ref[...]
    xg = jnp.dot(xb, g_ref[...], preferred_element_type=jnp.float32)
    xg_ref[...] = _pack_cols(xg)
    r_ref[...] = jnp.dot(xb, root_ref[...], preferred_element_type=jnp.float32)


def _pre(x_p, g, root):
    blk = 320
    grid = R // blk
    return pl.pallas_call(
        _pre_body,
        grid=(grid,),
        in_specs=[pl.BlockSpec((blk, D), lambda i: (i, 0)),
                  pl.BlockSpec((D, K * D), lambda i: (0, 0)),
                  pl.BlockSpec((D, D), lambda i: (0, 0))],
        out_specs=[pl.BlockSpec((blk, GW * 2), lambda i: (i, 0)),
                   pl.BlockSpec((blk, D), lambda i: (i, 0))],
        out_shape=[jax.ShapeDtypeStruct((R, GW * 2), jnp.bfloat16),
                   jax.ShapeDtypeStruct((R, D), jnp.float32)],
    )(x_p, g, root)


def _sc_pass(xg, sd3, wx_flat, with_cnt):
    mesh = plsc.VectorSubcoreMesh(core_axis_name="c", subcore_axis_name="s")

    ch = 32 if with_cnt else 64
    nchunk = E_PER_W // ch
    nblk = nchunk // BCH

    out_type = [jax.ShapeDtypeStruct((NC, ACC_R, D), jnp.float32)]
    scratch = [
        pltpu.VMEM((BCH, 2, ch), jnp.int32),
        pltpu.VMEM((ch * WXI,), jnp.int32),
        pltpu.VMEM((ch, GW), jnp.int32),
        pltpu.VMEM((ch, GW), jnp.int32),
        pltpu.VMEM((ch, D), jnp.float32),
        pltpu.VMEM_SHARED((ACC_R, D), jnp.float32),
        pltpu.SemaphoreType.DMA,
        pltpu.SemaphoreType.DMA,
    ]
    if with_cnt:
        scratch += [
            pltpu.VMEM((CROWS, D), jnp.float32),
            pltpu.VMEM((CROWS,), jnp.int32),
        ]

    @functools.partial(
        pl.kernel, out_type=out_type, mesh=mesh, scratch_types=scratch,
        compiler_params=pltpu.CompilerParams(needs_layout_passes=False))
    def body(xg_hbm, sd_hbm, wx_hbm, *rest):
        if with_cnt:
            (out_hbm, sd_blk, wx_v, rows0, rows1, msg_v, acc, sem0, sem1,
             cnt_loc, idx_v) = rest
        else:
            (out_hbm, sd_blk, wx_v, rows0, rows1, msg_v, acc, sem0,
             sem1) = rest
        c = lax.axis_index("c")
        s = lax.axis_index("s")
        wid = c * NS + s
        chunk0 = wid * nchunk

        zero16 = jnp.zeros((16,), jnp.float32)
        lane = lax.iota(jnp.int32, 16)

        def zrow(i, _):
            for j in range(D // 16):
                msg_v[i, pl.ds(j * 16, 16)] = zero16
            return 0
        lax.fori_loop(0, ch, zrow, 0)
        for z in range(ROWS_PER_TILE // ch):
            pltpu.sync_copy(msg_v, acc.at[pl.ds(s * ROWS_PER_TILE + z * ch, ch)])
        if ROWS_PER_TILE % ch:
            pltpu.sync_copy(
                msg_v.at[pl.ds(0, ROWS_PER_TILE % ch)],
                acc.at[pl.ds(s * ROWS_PER_TILE + (ROWS_PER_TILE // ch) * ch,
                             ROWS_PER_TILE % ch)])
        if with_cnt:
            def zcnt(i, _):
                for j in range(D // 16):
                    cnt_loc[i, pl.ds(j * 16, 16)] = zero16
                return 0
            lax.fori_loop(0, CROWS, zcnt, 0)

            def ziota(t, _):
                idx_v[pl.ds(t * 16, 16)] = R + t * 16 + lane
                return 0
            lax.fori_loop(0, CROWS // 16, ziota, 0)
        plsc.subcore_barrier()

        def gather(cl, rows, sem):
            pltpu.make_async_copy(
                xg_hbm.at[sd_blk.at[cl, 0]], rows, sem).start()

        def do_chunk(b, cl, rows, sem):
            gchunk = chunk0 + b * BCH + cl
            pltpu.sync_copy(
                wx_hbm.at[pl.ds(gchunk * ch * WXI, ch * WXI)], wx_v)
            pltpu.make_async_copy(
                xg_hbm.at[sd_blk.at[cl, 0]], rows, sem).wait()

            def edge(i, _):
                w01 = plsc.bitcast(wx_v[pl.ds(i * WXI, 16)], jnp.bfloat16)
                a0, a1 = plsc.unpack(w01, format=plsc.PackFormat.INTERLEAVED)
                w22 = plsc.bitcast(wx_v[pl.ds(i * WXI + 16, 16)],
                                   jnp.bfloat16)
                a2, _unused = plsc.unpack(
                    w22, format=plsc.PackFormat.INTERLEAVED)
                for m in range(K * D // 96):
                    v0 = plsc.bitcast(rows[i, pl.ds(16 * m, 16)],
                                      jnp.bfloat16)
                    v1 = plsc.bitcast(rows[i, pl.ds(D // 2 + 16 * m, 16)],
                                      jnp.bfloat16)
                    v2 = plsc.bitcast(rows[i, pl.ds(D + 16 * m, 16)],
                                      jnp.bfloat16)
                    p0 = plsc.unpack(v0, format=plsc.PackFormat.INTERLEAVED)
                    p1 = plsc.unpack(v1, format=plsc.PackFormat.INTERLEAVED)
                    p2 = plsc.unpack(v2, format=plsc.PackFormat.INTERLEAVED)
                    msg_v[i, pl.ds(32 * m, 16)] = (
                        a0 * p0[0] + a1 * p1[0] + a2 * p2[0])
                    msg_v[i, pl.ds(32 * m + 16, 16)] = (
                        a0 * p0[1] + a1 * p1[1] + a2 * p2[1])
                return 0
            lax.fori_loop(0, ch, edge, 0)

            if with_cnt:
                ones16 = jnp.ones((16,), jnp.float32)

                def hgrp(t, _):
                    dstg = sd_blk[cl, 1, pl.ds(t * 16, 16)]
                    plsc.addupdate_scatter(
                        cnt_loc, [dstg // D, dstg % D], ones16)
                    return 0
                lax.fori_loop(0, ch // 16, hgrp, 0)

            pltpu.sync_copy(msg_v, acc.at[sd_blk.at[cl, 1]], add=True)

        def block(b, _):
            pltpu.sync_copy(sd_hbm.at[pl.ds(chunk0 + b * BCH, BCH)], sd_blk)
            gather(0, rows0, sem0)

            def pair(j2, _):
                gather(2 * j2 + 1, rows1, sem1)
                do_chunk(b, 2 * j2, rows0, sem0)

                @pl.when(2 * j2 + 2 < BCH)
                def _():
                    gather(2 * j2 + 2, rows0, sem0)
                do_chunk(b, 2 * j2 + 1, rows1, sem1)
                return 0
            lax.fori_loop(0, BCH // 2, pair, 0)
            return 0
        lax.fori_loop(0, nblk, block, 0)

        plsc.subcore_barrier()
        if with_cnt:
            pltpu.sync_copy(cnt_loc, acc.at[idx_v], add=True)
            plsc.subcore_barrier()
        pltpu.sync_copy(acc.at[pl.ds(s * ROWS_PER_TILE, ROWS_PER_TILE)],
                        out_hbm.at[c, pl.ds(s * ROWS_PER_TILE, ROWS_PER_TILE)])

    return body(xg, sd3, wx_flat)


def _silu(y):
    return y * (1.0 / (1.0 + jnp.exp(-y)))


def _aggr(p_ref, cnt_ref):
    ssum = p_ref[0] + p_ref[1]
    cnt = cnt_ref[0, :] + cnt_ref[1, :]
    return ssum / jnp.maximum(cnt, 1.0)[:, None]


def _epi1_body(p_ref, cnt_ref, r1_ref, b1_ref, g2_ref, root2_ref,
               xg2_ref, r2_ref):
    y = _aggr(p_ref, cnt_ref) + r1_ref[...] + b1_ref[...][None, :]
    y = _silu(y)
    xg2 = jnp.dot(y, g2_ref[...], preferred_element_type=jnp.float32)
    xg2_ref[...] = _pack_cols(xg2)
    r2_ref[...] = jnp.dot(y, root2_ref[...], preferred_element_type=jnp.float32)


def _epi1(p, cnt, r1, b1, g2, root2):
    blk = 512
    grid = R // blk
    return pl.pallas_call(
        _epi1_body,
        grid=(grid,),
        in_specs=[pl.BlockSpec((NC, blk, D), lambda i: (0, i, 0)),
                  pl.BlockSpec((NC, blk), lambda i: (0, i)),
                  pl.BlockSpec((blk, D), lambda i: (i, 0)),
                  pl.BlockSpec((D,), lambda i: (0,)),
                  pl.BlockSpec((D, K * D), lambda i: (0, 0)),
                  pl.BlockSpec((D, D), lambda i: (0, 0))],
        out_specs=[pl.BlockSpec((blk, GW * 2), lambda i: (i, 0)),
                   pl.BlockSpec((blk, D), lambda i: (i, 0))],
        out_shape=[jax.ShapeDtypeStruct((R, GW * 2), jnp.bfloat16),
                   jax.ShapeDtypeStruct((R, D), jnp.float32)],
    )(p, cnt, r1, b1, g2, root2)


def _epi2_body(p_ref, cnt_ref, r2_ref, b2_ref, x_ref, ipv_ref, out_ref):
    y = _aggr(p_ref, cnt_ref) + r2_ref[...] + b2_ref[...][None, :]
    y = jnp.take_along_axis(
        y, jnp.broadcast_to(ipv_ref[...][None, :], y.shape), axis=1)
    out_ref[...] = _silu(y + x_ref[...])


def _epi2(p, cnt, r2, b2, x_p, ipv):
    blk = 512
    grid = R // blk
    return pl.pallas_call(
        _epi2_body,
        grid=(grid,),
        in_specs=[pl.BlockSpec((NC, blk, D), lambda i: (0, i, 0)),
                  pl.BlockSpec((NC, blk), lambda i: (0, i)),
                  pl.BlockSpec((blk, D), lambda i: (i, 0)),
                  pl.BlockSpec((D,), lambda i: (0,)),
                  pl.BlockSpec((blk, D), lambda i: (i, 0)),
                  pl.BlockSpec((D,), lambda i: (0,))],
        out_specs=pl.BlockSpec((blk, D), lambda i: (i, 0)),
        out_shape=jax.ShapeDtypeStruct((R, D), jnp.float32),
    )(p, cnt, r2, b2, x_p, ipv)


def kernel(x, edge_index, edge_attr, g1, mu1, sigma1, root1, bias1,
           g2, mu2, sigma2, root2, bias2):
    e = edge_attr.shape[0]
    pad = E_PAD - e
    src_p = jnp.concatenate([edge_index[0], jnp.zeros((pad,), jnp.int32)])
    dst_p = jnp.concatenate([edge_index[1], jnp.full((pad,), TRASH, jnp.int32)])
    attr_p = jnp.concatenate(
        [edge_attr, jnp.zeros((pad, D_ATTR), jnp.float32)])
    x_p = jnp.concatenate([x, jnp.zeros((R - N, D), jnp.float32)])

    wx1 = _wx_i32(_edge_weights(attr_p, mu1, sigma1))
    wx2 = _wx_i32(_edge_weights(attr_p, mu2, sigma2))

    def sd(ch):
        return jnp.stack(
            [src_p.reshape(-1, ch), dst_p.reshape(-1, ch)], axis=1)

    pv = jnp.asarray(PV)
    root1_p = root1[:, pv]
    bias1_p = bias1[pv]
    g2_p = g2[pv, :]
    root2_p = root2[pv][:, pv]
    bias2_p = bias2[pv]

    xg1, r1 = _pre(x_p, g1, root1_p)
    full1 = _sc_pass(_as_i32(xg1), sd(32), wx1, with_cnt=True)[0]
    p1 = full1[:, :R]
    cnt = full1[:, R:R + CROWS].reshape(NC, R)
    xg2, r2 = _epi1(p1, cnt, r1, bias1_p, g2_p, root2_p)
    p2 = _sc_pass(_as_i32(xg2), sd(64), wx2, with_cnt=False)[0][:, :R]
    out = _epi2(p2, cnt, r2, bias2_p, x_p, jnp.asarray(IPV))
    return out[:N]


# speedup vs baseline: 2.4408x; 2.4408x over previous
import functools

import jax
import jax.numpy as jnp
import numpy as np
from jax import lax
from jax.experimental import pallas as pl
from jax.experimental.pallas import tpu as pltpu
from jax.experimental.pallas import tpu_sc as plsc

N = 10000
D = 128
K = 3
D_ATTR = 16
EPS = 1e-15

NC = 2
NS = 16
NW = NC * NS
E_PER_W = 10240
E_PAD = NW * E_PER_W
R = 10240
CROWS = R // D
ACC_R = R + CROWS + 48
ROWS_PER_TILE = ACC_R // NS
TRASH = N + 50
WXW = K * 16
BCH = 8
GW = 256

PV = np.concatenate(
    [32 * j + np.concatenate([np.arange(0, 32, 2), np.arange(1, 32, 2)])
     for j in range(4)]).astype(np.int32)
IPV = np.argsort(PV).astype(np.int32)


def _pack_cols(xg):
    b = xg.shape[0]
    return jnp.concatenate(
        [xg.astype(jnp.bfloat16),
         jnp.zeros((b, GW * 2 - K * D), jnp.bfloat16)], axis=1)


def _as_i32(xgp):
    return lax.bitcast_convert_type(xgp.reshape(R, GW, 2), jnp.int32)


def _w_body(attr_ref, mu1_ref, s1_ref, mu2_ref, s2_ref, wx1_ref, wx2_ref):
    a = attr_ref[...]
    for mu_ref, s_ref, out in ((mu1_ref, s1_ref, wx1_ref),
                               (mu2_ref, s2_ref, wx2_ref)):
        cols = []
        for k in range(K):
            mu = mu_ref[k, :]
            s2 = EPS + s_ref[k, :] ** 2
            g = -0.5 * (a - mu[None, :]) ** 2 / s2[None, :]
            w = jnp.exp(jnp.sum(g, axis=1))
            cols.append(jnp.broadcast_to(w[:, None], (w.shape[0], 16)))
        out[...] = jnp.concatenate(cols, axis=1)


def _edge_weights(edge_attr_p, mu1, sigma1, mu2, sigma2):
    blk = 4096
    grid = E_PAD // blk
    outs = [jax.ShapeDtypeStruct((E_PAD, WXW), jnp.float32)] * 2
    small = pl.BlockSpec((K, D_ATTR), lambda i: (0, 0))
    return pl.pallas_call(
        _w_body,
        grid=(grid,),
        in_specs=[pl.BlockSpec((blk, D_ATTR), lambda i: (i, 0)),
                  small, small, small, small],
        out_specs=[pl.BlockSpec((blk, WXW), lambda i: (i, 0))] * 2,
        out_shape=outs,
    )(edge_attr_p, mu1, sigma1, mu2, sigma2)


def _pre_body(x_ref, g_ref, root_ref, xg_ref, r_ref):
    xb = x_ref[...]
    xg = jnp.dot(xb, g_ref[...], preferred_element_type=jnp.float32)
    xg_ref[...] = _pack_cols(xg)
    r_ref[...] = jnp.dot(xb, root_ref[...], preferred_element_type=jnp.float32)


def _pre(x_p, g, root):
    blk = 320
    grid = R // blk
    return pl.pallas_call(
        _pre_body,
        grid=(grid,),
        in_specs=[pl.BlockSpec((blk, D), lambda i: (i, 0)),
                  pl.BlockSpec((D, K * D), lambda i: (0, 0)),
                  pl.BlockSpec((D, D), lambda i: (0, 0))],
        out_specs=[pl.BlockSpec((blk, GW * 2), lambda i: (i, 0)),
                   pl.BlockSpec((blk, D), lambda i: (i, 0))],
        out_shape=[jax.ShapeDtypeStruct((R, GW * 2), jnp.bfloat16),
                   jax.ShapeDtypeStruct((R, D), jnp.float32)],
    )(x_p, g, root)


def _sc_pass(xg, sd3, wx_flat, with_cnt):
    mesh = plsc.VectorSubcoreMesh(core_axis_name="c", subcore_axis_name="s")

    ch = 32 if with_cnt else 64
    nchunk = E_PER_W // ch
    nblk = nchunk // BCH

    out_type = [jax.ShapeDtypeStruct((NC, ACC_R, D), jnp.float32)]
    scratch = [
        pltpu.VMEM((BCH, 2, ch), jnp.int32),
        pltpu.VMEM((ch * WXW,), jnp.float32),
        pltpu.VMEM((ch, GW), jnp.int32),
        pltpu.VMEM((ch, GW), jnp.int32),
        pltpu.VMEM((ch, D), jnp.float32),
        pltpu.VMEM_SHARED((ACC_R, D), jnp.float32),
        pltpu.SemaphoreType.DMA,
        pltpu.SemaphoreType.DMA,
    ]
    if with_cnt:
        scratch += [
            pltpu.VMEM((CROWS, D), jnp.float32),
            pltpu.VMEM((CROWS,), jnp.int32),
        ]

    @functools.partial(
        pl.kernel, out_type=out_type, mesh=mesh, scratch_types=scratch,
        compiler_params=pltpu.CompilerParams(needs_layout_passes=False))
    def body(xg_hbm, sd_hbm, wx_hbm, *rest):
        if with_cnt:
            (out_hbm, sd_blk, wx_v, rows0, rows1, msg_v, acc, sem0, sem1,
             cnt_loc, idx_v) = rest
        else:
            (out_hbm, sd_blk, wx_v, rows0, rows1, msg_v, acc, sem0,
             sem1) = rest
        c = lax.axis_index("c")
        s = lax.axis_index("s")
        wid = c * NS + s
        chunk0 = wid * nchunk

        zero16 = jnp.zeros((16,), jnp.float32)
        lane = lax.iota(jnp.int32, 16)

        def zrow(i, _):
            for j in range(D // 16):
                msg_v[i, pl.ds(j * 16, 16)] = zero16
            return 0
        lax.fori_loop(0, ch, zrow, 0)
        for z in range(ROWS_PER_TILE // ch):
            pltpu.sync_copy(msg_v, acc.at[pl.ds(s * ROWS_PER_TILE + z * ch, ch)])
        if ROWS_PER_TILE % ch:
            pltpu.sync_copy(
                msg_v.at[pl.ds(0, ROWS_PER_TILE % ch)],
                acc.at[pl.ds(s * ROWS_PER_TILE + (ROWS_PER_TILE // ch) * ch,
                             ROWS_PER_TILE % ch)])
        if with_cnt:
            def zcnt(i, _):
                for j in range(D // 16):
                    cnt_loc[i, pl.ds(j * 16, 16)] = zero16
                return 0
            lax.fori_loop(0, CROWS, zcnt, 0)

            def ziota(t, _):
                idx_v[pl.ds(t * 16, 16)] = R + t * 16 + lane
                return 0
            lax.fori_loop(0, CROWS // 16, ziota, 0)
        plsc.subcore_barrier()

        def gather(cl, rows, sem):
            pltpu.make_async_copy(
                xg_hbm.at[sd_blk.at[cl, 0]], rows, sem).start()

        def do_chunk(b, cl, rows, sem):
            gchunk = chunk0 + b * BCH + cl
            pltpu.sync_copy(
                wx_hbm.at[pl.ds(gchunk * ch * WXW, ch * WXW)], wx_v)
            pltpu.make_async_copy(
                xg_hbm.at[sd_blk.at[cl, 0]], rows, sem).wait()

            def edge(i, _):
                a0 = wx_v[pl.ds(i * WXW, 16)]
                a1 = wx_v[pl.ds(i * WXW + 16, 16)]
                a2 = wx_v[pl.ds(i * WXW + 32, 16)]
                for m in range(K * D // 96):
                    v0 = plsc.bitcast(rows[i, pl.ds(16 * m, 16)],
                                      jnp.bfloat16)
                    v1 = plsc.bitcast(rows[i, pl.ds(D // 2 + 16 * m, 16)],
                                      jnp.bfloat16)
                    v2 = plsc.bitcast(rows[i, pl.ds(D + 16 * m, 16)],
                                      jnp.bfloat16)
                    p0 = plsc.unpack(v0, format=plsc.PackFormat.INTERLEAVED)
                    p1 = plsc.unpack(v1, format=plsc.PackFormat.INTERLEAVED)
                    p2 = plsc.unpack(v2, format=plsc.PackFormat.INTERLEAVED)
                    msg_v[i, pl.ds(32 * m, 16)] = (
                        a0 * p0[0] + a1 * p1[0] + a2 * p2[0])
                    msg_v[i, pl.ds(32 * m + 16, 16)] = (
                        a0 * p0[1] + a1 * p1[1] + a2 * p2[1])
                return 0
            lax.fori_loop(0, ch, edge, 0)

            if with_cnt:
                ones16 = jnp.ones((16,), jnp.float32)

                def hgrp(t, _):
                    dstg = sd_blk[cl, 1, pl.ds(t * 16, 16)]
                    plsc.addupdate_scatter(
                        cnt_loc, [dstg // D, dstg % D], ones16)
                    return 0
                lax.fori_loop(0, ch // 16, hgrp, 0)

            pltpu.sync_copy(msg_v, acc.at[sd_blk.at[cl, 1]], add=True)

        def block(b, _):
            pltpu.sync_copy(sd_hbm.at[pl.ds(chunk0 + b * BCH, BCH)], sd_blk)
            gather(0, rows0, sem0)

            def pair(j2, _):
                gather(2 * j2 + 1, rows1, sem1)
                do_chunk(b, 2 * j2, rows0, sem0)

                @pl.when(2 * j2 + 2 < BCH)
                def _():
                    gather(2 * j2 + 2, rows0, sem0)
                do_chunk(b, 2 * j2 + 1, rows1, sem1)
                return 0
            lax.fori_loop(0, BCH // 2, pair, 0)
            return 0
        lax.fori_loop(0, nblk, block, 0)

        plsc.subcore_barrier()
        if with_cnt:
            pltpu.sync_copy(cnt_loc, acc.at[idx_v], add=True)
            plsc.subcore_barrier()
        pltpu.sync_copy(acc.at[pl.ds(s * ROWS_PER_TILE, ROWS_PER_TILE)],
                        out_hbm.at[c, pl.ds(s * ROWS_PER_TILE, ROWS_PER_TILE)])

    return body(xg, sd3, wx_flat)


def _silu(y):
    return y * (1.0 / (1.0 + jnp.exp(-y)))


def _aggr(p_ref, cnt_ref):
    ssum = p_ref[0] + p_ref[1]
    cnt = cnt_ref[0, :] + cnt_ref[1, :]
    return ssum / jnp.maximum(cnt, 1.0)[:, None]


def _epi1_body(p_ref, cnt_ref, r1_ref, b1_ref, g2_ref, root2_ref,
               xg2_ref, r2_ref):
    y = _aggr(p_ref, cnt_ref) + r1_ref[...] + b1_ref[...][None, :]
    y = _silu(y)
    xg2 = jnp.dot(y, g2_ref[...], preferred_element_type=jnp.float32)
    xg2_ref[...] = _pack_cols(xg2)
    r2_ref[...] = jnp.dot(y, root2_ref[...], preferred_element_type=jnp.float32)


def _epi1(p, cnt, r1, b1, g2, root2):
    blk = 512
    grid = R // blk
    return pl.pallas_call(
        _epi1_body,
        grid=(grid,),
        in_specs=[pl.BlockSpec((NC, blk, D), lambda i: (0, i, 0)),
                  pl.BlockSpec((NC, blk), lambda i: (0, i)),
                  pl.BlockSpec((blk, D), lambda i: (i, 0)),
                  pl.BlockSpec((D,), lambda i: (0,)),
                  pl.BlockSpec((D, K * D), lambda i: (0, 0)),
                  pl.BlockSpec((D, D), lambda i: (0, 0))],
        out_specs=[pl.BlockSpec((blk, GW * 2), lambda i: (i, 0)),
                   pl.BlockSpec((blk, D), lambda i: (i, 0))],
        out_shape=[jax.ShapeDtypeStruct((R, GW * 2), jnp.bfloat16),
                   jax.ShapeDtypeStruct((R, D), jnp.float32)],
    )(p, cnt, r1, b1, g2, root2)


def _epi2_body(p_ref, cnt_ref, r2_ref, b2_ref, x_ref, ipv_ref, out_ref):
    y = _aggr(p_ref, cnt_ref) + r2_ref[...] + b2_ref[...][None, :]
    y = jnp.take_along_axis(
        y, jnp.broadcast_to(ipv_ref[...][None, :], y.shape), axis=1)
    out_ref[...] = _silu(y + x_ref[...])


def _epi2(p, cnt, r2, b2, x_p, ipv):
    blk = 512
    grid = R // blk
    return pl.pallas_call(
        _epi2_body,
        grid=(grid,),
        in_specs=[pl.BlockSpec((NC, blk, D), lambda i: (0, i, 0)),
                  pl.BlockSpec((NC, blk), lambda i: (0, i)),
                  pl.BlockSpec((blk, D), lambda i: (i, 0)),
                  pl.BlockSpec((D,), lambda i: (0,)),
                  pl.BlockSpec((blk, D), lambda i: (i, 0)),
                  pl.BlockSpec((D,), lambda i: (0,))],
        out_specs=pl.BlockSpec((blk, D), lambda i: (i, 0)),
        out_shape=jax.ShapeDtypeStruct((R, D), jnp.float32),
    )(p, cnt, r2, b2, x_p, ipv)


def kernel(x, edge_index, edge_attr, g1, mu1, sigma1, root1, bias1,
           g2, mu2, sigma2, root2, bias2):
    e = edge_attr.shape[0]
    pad = E_PAD - e
    src_p = jnp.concatenate([edge_index[0], jnp.zeros((pad,), jnp.int32)])
    dst_p = jnp.concatenate([edge_index[1], jnp.full((pad,), TRASH, jnp.int32)])
    attr_p = jnp.concatenate(
        [edge_attr, jnp.zeros((pad, D_ATTR), jnp.float32)])
    x_p = jnp.concatenate([x, jnp.zeros((R - N, D), jnp.float32)])

    wx1, wx2 = _edge_weights(attr_p, mu1, sigma1, mu2, sigma2)
    wx1, wx2 = wx1.reshape(-1), wx2.reshape(-1)

    def sd(ch):
        return jnp.stack(
            [src_p.reshape(-1, ch), dst_p.reshape(-1, ch)], axis=1)

    pv = jnp.asarray(PV)
    root1_p = root1[:, pv]
    bias1_p = bias1[pv]
    g2_p = g2[pv, :]
    root2_p = root2[pv][:, pv]
    bias2_p = bias2[pv]

    xg1, r1 = _pre(x_p, g1, root1_p)
    full1 = _sc_pass(_as_i32(xg1), sd(32), wx1, with_cnt=True)[0]
    p1 = full1[:, :R]
    cnt = full1[:, R:R + CROWS].reshape(NC, R)
    xg2, r2 = _epi1(p1, cnt, r1, bias1_p, g2_p, root2_p)
    p2 = _sc_pass(_as_i32(xg2), sd(64), wx2, with_cnt=False)[0][:, :R]
    out = _epi2(p2, cnt, r2, bias2_p, x_p, jnp.asarray(IPV))
    return out[:N]


# speedup vs baseline: 2.4685x; 1.0113x over previous
import functools

import jax
import jax.numpy as jnp
import numpy as np
from jax import lax
from jax.experimental import pallas as pl
from jax.experimental.pallas import tpu as pltpu
from jax.experimental.pallas import tpu_sc as plsc

N = 10000
D = 128
K = 3
D_ATTR = 16
EPS = 1e-15

NC = 2
NS = 16
NW = NC * NS
E_PER_W = 10240
E_PAD = NW * E_PER_W
R = 10240
CROWS = R // D
ACC_R = R + CROWS + 48
ROWS_PER_TILE = ACC_R // NS
TRASH = N + 50
WXW = K * 16
BCH = 8
GW = 256

PV = np.concatenate(
    [32 * j + np.concatenate([np.arange(0, 32, 2), np.arange(1, 32, 2)])
     for j in range(4)]).astype(np.int32)
IPV = np.argsort(PV).astype(np.int32)


def _pack_cols(xg):
    b = xg.shape[0]
    return jnp.concatenate(
        [xg.astype(jnp.bfloat16),
         jnp.zeros((b, GW * 2 - K * D), jnp.bfloat16)], axis=1)


def _as_i32(xgp):
    return lax.bitcast_convert_type(xgp.reshape(R, GW, 2), jnp.int32)


def _w_body(attr_ref, mu1_ref, s1_ref, mu2_ref, s2_ref, wx1_ref, wx2_ref):
    a = attr_ref[...]
    for mu_ref, s_ref, out in ((mu1_ref, s1_ref, wx1_ref),
                               (mu2_ref, s2_ref, wx2_ref)):
        cols = []
        for k in range(K):
            mu = mu_ref[k, :]
            s2 = EPS + s_ref[k, :] ** 2
            g = -0.5 * (a - mu[None, :]) ** 2 / s2[None, :]
            w = jnp.exp(jnp.sum(g, axis=1))
            cols.append(jnp.broadcast_to(w[:, None], (w.shape[0], 16)))
        out[...] = jnp.concatenate(cols, axis=1)


def _edge_weights(edge_attr_p, mu1, sigma1, mu2, sigma2):
    blk = 4096
    grid = E_PAD // blk
    outs = [jax.ShapeDtypeStruct((E_PAD, WXW), jnp.float32)] * 2
    small = pl.BlockSpec((K, D_ATTR), lambda i: (0, 0))
    return pl.pallas_call(
        _w_body,
        grid=(grid,),
        in_specs=[pl.BlockSpec((blk, D_ATTR), lambda i: (i, 0)),
                  small, small, small, small],
        out_specs=[pl.BlockSpec((blk, WXW), lambda i: (i, 0))] * 2,
        out_shape=outs,
    )(edge_attr_p, mu1, sigma1, mu2, sigma2)


def _pre_body(x_ref, g_ref, root_ref, xg_ref, r_ref):
    xb = x_ref[...]
    xg = jnp.dot(xb, g_ref[...], preferred_element_type=jnp.float32)
    xg_ref[...] = _pack_cols(xg)
    r_ref[...] = jnp.dot(xb, root_ref[...], preferred_element_type=jnp.float32)


def _pre(x_p, g, root):
    blk = 320
    grid = R // blk
    return pl.pallas_call(
        _pre_body,
        grid=(grid,),
        in_specs=[pl.BlockSpec((blk, D), lambda i: (i, 0)),
                  pl.BlockSpec((D, K * D), lambda i: (0, 0)),
                  pl.BlockSpec((D, D), lambda i: (0, 0))],
        out_specs=[pl.BlockSpec((blk, GW * 2), lambda i: (i, 0)),
                   pl.BlockSpec((blk, D), lambda i: (i, 0))],
        out_shape=[jax.ShapeDtypeStruct((R, GW * 2), jnp.bfloat16),
                   jax.ShapeDtypeStruct((R, D), jnp.float32)],
    )(x_p, g, root)


def _sc_pass(xg, sd3, wx_flat, with_cnt):
    mesh = plsc.VectorSubcoreMesh(core_axis_name="c", subcore_axis_name="s")

    ch = 32
    nchunk = E_PER_W // ch
    nblk = nchunk // BCH

    out_type = [jax.ShapeDtypeStruct((NC, ACC_R, D), jnp.float32)]
    scratch = (
        [pltpu.VMEM((BCH, 2, ch), jnp.int32)]
        + [pltpu.VMEM((ch * WXW,), jnp.float32)] * 2
        + [pltpu.VMEM((ch, GW), jnp.int32)] * 2
        + [pltpu.VMEM((ch, D), jnp.float32)] * 3
        + [pltpu.VMEM_SHARED((ACC_R, D), jnp.float32)]
        + [pltpu.SemaphoreType.DMA] * 7
    )
    if with_cnt:
        scratch += [
            pltpu.VMEM((CROWS, D), jnp.float32),
            pltpu.VMEM((CROWS,), jnp.int32),
        ]

    @functools.partial(
        pl.kernel, out_type=out_type, mesh=mesh, scratch_types=scratch,
        compiler_params=pltpu.CompilerParams(needs_layout_passes=False))
    def body(xg_hbm, sd_hbm, wx_hbm, *rest):
        out_hbm = rest[0]
        sd_blk = rest[1]
        wx_b = rest[2:4]
        rows_b = rest[4:6]
        msg_b = rest[6:9]
        acc = rest[9]
        gsem = rest[10:12]
        wsem = rest[12:14]
        ssem = rest[14:17]
        if with_cnt:
            cnt_loc, idx_v = rest[17], rest[18]
        c = lax.axis_index("c")
        s = lax.axis_index("s")
        wid = c * NS + s
        chunk0 = wid * nchunk

        zero16 = jnp.zeros((16,), jnp.float32)
        lane = lax.iota(jnp.int32, 16)

        def zrow(i, _):
            for mb in msg_b:
                for j in range(D // 16):
                    mb[i, pl.ds(j * 16, 16)] = zero16
            return 0
        lax.fori_loop(0, ch, zrow, 0)
        for z in range(ROWS_PER_TILE // ch):
            pltpu.sync_copy(msg_b[0],
                            acc.at[pl.ds(s * ROWS_PER_TILE + z * ch, ch)])
        if ROWS_PER_TILE % ch:
            pltpu.sync_copy(
                msg_b[0].at[pl.ds(0, ROWS_PER_TILE % ch)],
                acc.at[pl.ds(s * ROWS_PER_TILE + (ROWS_PER_TILE // ch) * ch,
                             ROWS_PER_TILE % ch)])
        if with_cnt:
            def zcnt(i, _):
                for j in range(D // 16):
                    cnt_loc[i, pl.ds(j * 16, 16)] = zero16
                return 0
            lax.fori_loop(0, CROWS, zcnt, 0)

            def ziota(t, _):
                idx_v[pl.ds(t * 16, 16)] = R + t * 16 + lane
                return 0
            lax.fori_loop(0, CROWS // 16, ziota, 0)
        plsc.subcore_barrier()

        def issue(b, cl):
            gchunk = chunk0 + b * BCH + cl
            pltpu.make_async_copy(
                xg_hbm.at[sd_blk.at[cl, 0]], rows_b[cl % 2],
                gsem[cl % 2]).start()
            pltpu.make_async_copy(
                wx_hbm.at[pl.ds(gchunk * ch * WXW, ch * WXW)],
                wx_b[cl % 2], wsem[cl % 2]).start()

        def do_chunk(b, cl):
            gchunk = chunk0 + b * BCH + cl
            rows = rows_b[cl % 2]
            wx_v = wx_b[cl % 2]
            msg_v = msg_b[cl % 3]
            pltpu.make_async_copy(
                wx_hbm.at[pl.ds(gchunk * ch * WXW, ch * WXW)],
                wx_v, wsem[cl % 2]).wait()
            pltpu.make_async_copy(
                xg_hbm.at[sd_blk.at[cl, 0]], rows, gsem[cl % 2]).wait()

            def edge(i, _):
                a0 = wx_v[pl.ds(i * WXW, 16)]
                a1 = wx_v[pl.ds(i * WXW + 16, 16)]
                a2 = wx_v[pl.ds(i * WXW + 32, 16)]
                for m in range(K * D // 96):
                    v0 = plsc.bitcast(rows[i, pl.ds(16 * m, 16)],
                                      jnp.bfloat16)
                    v1 = plsc.bitcast(rows[i, pl.ds(D // 2 + 16 * m, 16)],
                                      jnp.bfloat16)
                    v2 = plsc.bitcast(rows[i, pl.ds(D + 16 * m, 16)],
                                      jnp.bfloat16)
                    p0 = plsc.unpack(v0, format=plsc.PackFormat.INTERLEAVED)
                    p1 = plsc.unpack(v1, format=plsc.PackFormat.INTERLEAVED)
                    p2 = plsc.unpack(v2, format=plsc.PackFormat.INTERLEAVED)
                    msg_v[i, pl.ds(32 * m, 16)] = (
                        a0 * p0[0] + a1 * p1[0] + a2 * p2[0])
                    msg_v[i, pl.ds(32 * m + 16, 16)] = (
                        a0 * p0[1] + a1 * p1[1] + a2 * p2[1])
                return 0
            lax.fori_loop(0, ch, edge, 0)

            if with_cnt:
                ones16 = jnp.ones((16,), jnp.float32)

                def hgrp(t, _):
                    dstg = sd_blk[cl, 1, pl.ds(t * 16, 16)]
                    plsc.addupdate_scatter(
                        cnt_loc, [dstg // D, dstg % D], ones16)
                    return 0
                lax.fori_loop(0, ch // 16, hgrp, 0)

            pltpu.sync_copy(msg_v, acc.at[sd_blk.at[cl, 1]], add=True)

        def wait_scatter(cl):
            pltpu.make_async_copy(
                msg_b[cl % 3], acc.at[sd_blk.at[cl, 1]], ssem[cl % 3]).wait()

        def block(b, _):
            pltpu.sync_copy(sd_hbm.at[pl.ds(chunk0 + b * BCH, BCH)], sd_blk)
            issue(b, 0)
            for cl in range(BCH):
                if cl + 1 < BCH:
                    issue(b, cl + 1)
                do_chunk(b, cl)
            return 0
        lax.fori_loop(0, nblk, block, 0)

        plsc.subcore_barrier()
        if with_cnt:
            pltpu.sync_copy(cnt_loc, acc.at[idx_v], add=True)
            plsc.subcore_barrier()
        pltpu.sync_copy(acc.at[pl.ds(s * ROWS_PER_TILE, ROWS_PER_TILE)],
                        out_hbm.at[c, pl.ds(s * ROWS_PER_TILE, ROWS_PER_TILE)])

    return body(xg, sd3, wx_flat)


def _silu(y):
    return y * (1.0 / (1.0 + jnp.exp(-y)))


def _aggr(p_ref, cnt_ref):
    ssum = p_ref[0] + p_ref[1]
    cnt = cnt_ref[0, :] + cnt_ref[1, :]
    return ssum / jnp.maximum(cnt, 1.0)[:, None]


def _epi1_body(p_ref, cnt_ref, r1_ref, b1_ref, g2_ref, root2_ref,
               xg2_ref, r2_ref):
    y = _aggr(p_ref, cnt_ref) + r1_ref[...] + b1_ref[...][None, :]
    y = _silu(y)
    xg2 = jnp.dot(y, g2_ref[...], preferred_element_type=jnp.float32)
    xg2_ref[...] = _pack_cols(xg2)
    r2_ref[...] = jnp.dot(y, root2_ref[...], preferred_element_type=jnp.float32)


def _epi1(p, cnt, r1, b1, g2, root2):
    blk = 512
    grid = R // blk
    return pl.pallas_call(
        _epi1_body,
        grid=(grid,),
        in_specs=[pl.BlockSpec((NC, blk, D), lambda i: (0, i, 0)),
                  pl.BlockSpec((NC, blk), lambda i: (0, i)),
                  pl.BlockSpec((blk, D), lambda i: (i, 0)),
                  pl.BlockSpec((D,), lambda i: (0,)),
                  pl.BlockSpec((D, K * D), lambda i: (0, 0)),
                  pl.BlockSpec((D, D), lambda i: (0, 0))],
        out_specs=[pl.BlockSpec((blk, GW * 2), lambda i: (i, 0)),
                   pl.BlockSpec((blk, D), lambda i: (i, 0))],
        out_shape=[jax.ShapeDtypeStruct((R, GW * 2), jnp.bfloat16),
                   jax.ShapeDtypeStruct((R, D), jnp.float32)],
    )(p, cnt, r1, b1, g2, root2)


def _epi2_body(p_ref, cnt_ref, r2_ref, b2_ref, x_ref, ipv_ref, out_ref):
    y = _aggr(p_ref, cnt_ref) + r2_ref[...] + b2_ref[...][None, :]
    y = jnp.take_along_axis(
        y, jnp.broadcast_to(ipv_ref[...][None, :], y.shape), axis=1)
    out_ref[...] = _silu(y + x_ref[...])


def _epi2(p, cnt, r2, b2, x_p, ipv):
    blk = 512
    grid = R // blk
    return pl.pallas_call(
        _epi2_body,
        grid=(grid,),
        in_specs=[pl.BlockSpec((NC, blk, D), lambda i: (0, i, 0)),
                  pl.BlockSpec((NC, blk), lambda i: (0, i)),
                  pl.BlockSpec((blk, D), lambda i: (i, 0)),
                  pl.BlockSpec((D,), lambda i: (0,)),
                  pl.BlockSpec((blk, D), lambda i: (i, 0)),
                  pl.BlockSpec((D,), lambda i: (0,))],
        out_specs=pl.BlockSpec((blk, D), lambda i: (i, 0)),
        out_shape=jax.ShapeDtypeStruct((R, D), jnp.float32),
    )(p, cnt, r2, b2, x_p, ipv)


def kernel(x, edge_index, edge_attr, g1, mu1, sigma1, root1, bias1,
           g2, mu2, sigma2, root2, bias2):
    e = edge_attr.shape[0]
    pad = E_PAD - e
    src_p = jnp.concatenate([edge_index[0], jnp.zeros((pad,), jnp.int32)])
    dst_p = jnp.concatenate([edge_index[1], jnp.full((pad,), TRASH, jnp.int32)])
    attr_p = jnp.concatenate(
        [edge_attr, jnp.zeros((pad, D_ATTR), jnp.float32)])
    x_p = jnp.concatenate([x, jnp.zeros((R - N, D), jnp.float32)])

    wx1, wx2 = _edge_weights(attr_p, mu1, sigma1, mu2, sigma2)
    wx1, wx2 = wx1.reshape(-1), wx2.reshape(-1)

    def sd(ch):
        return jnp.stack(
            [src_p.reshape(-1, ch), dst_p.reshape(-1, ch)], axis=1)

    pv = jnp.asarray(PV)
    root1_p = root1[:, pv]
    bias1_p = bias1[pv]
    g2_p = g2[pv, :]
    root2_p = root2[pv][:, pv]
    bias2_p = bias2[pv]

    xg1, r1 = _pre(x_p, g1, root1_p)
    full1 = _sc_pass(_as_i32(xg1), sd(32), wx1, with_cnt=True)[0]
    p1 = full1[:, :R]
    cnt = full1[:, R:R + CROWS].reshape(NC, R)
    xg2, r2 = _epi1(p1, cnt, r1, bias1_p, g2_p, root2_p)
    p2 = _sc_pass(_as_i32(xg2), sd(32), wx2, with_cnt=False)[0][:, :R]
    out = _epi2(p2, cnt, r2, bias2_p, x_p, jnp.asarray(IPV))
    return out[:N]
